# baseline (speedup 1.0000x reference)
import jax
import jax.numpy as jnp
from jax import lax
from jax.experimental import pallas as pl
from jax.experimental.pallas import tpu as pltpu
from jax.experimental.pallas import tpu_sc as plsc

B, S, D, E = 4, 8192, 2048, 64
VEC = 16
NC, NS = 2, 16
NW = NC * NS
WPB = NW // B

S_TC = 5632
CHUNK = 256
S_SC = S - S_TC
RPW = S_SC // WPB
R = 16
NCHUNK = RPW // R
NG = NCHUNK // 2


def _sc_sum(x_hbm, out_hbm, accA, accB, acc, idxA, idxB, semA, semB):
    wid = lax.axis_index("s") * NC + lax.axis_index("c")
    batch = wid // WPB
    slot = wid % WPB
    base = batch * S + S_TC + slot * RPW
    iota = lax.iota(jnp.int32, VEC)

    def set_start(c, acc_ref, idx_ref, sem, add):
        idx_ref[...] = base + c * R + iota
        pltpu.async_copy(x_hbm.at[idx_ref], acc_ref, sem, add=add)

    def wait(acc_ref, sem):
        pltpu.make_async_copy(x_hbm.at[idxA], acc_ref, sem).wait()

    set_start(0, accA, idxA, semA, False)
    set_start(1, accB, idxB, semB, False)

    def body(g, _):
        wait(accA, semA)
        set_start(2 * g, accA, idxA, semA, True)
        wait(accB, semB)
        set_start(2 * g + 1, accB, idxB, semB, True)
        return 0

    lax.fori_loop(1, NG, body, 0)
    wait(accA, semA)
    wait(accB, semB)

    @plsc.parallel_loop(0, D // VEC, unroll=2)
    def _red(j):
        col = j * VEC
        v = accA[0, pl.ds(col, VEC)]
        for r in range(1, R):
            v = v + accA[r, pl.ds(col, VEC)]
        for r in range(R):
            v = v + accB[r, pl.ds(col, VEC)]
        acc[0, pl.ds(col, VEC)] = v

    pltpu.sync_copy(acc, out_hbm.at[pl.ds(wid, 1)])


_sc_sum_call = pl.kernel(
    _sc_sum,
    out_type=jax.ShapeDtypeStruct((NW, D), jnp.float32),
    mesh=plsc.VectorSubcoreMesh(core_axis_name="c", subcore_axis_name="s"),
    scratch_types=[
        pltpu.VMEM((R, D), jnp.float32),
        pltpu.VMEM((R, D), jnp.float32),
        pltpu.VMEM((1, D), jnp.float32),
        pltpu.VMEM((VEC,), jnp.int32),
        pltpu.VMEM((VEC,), jnp.int32),
        pltpu.SemaphoreType.DMA,
        pltpu.SemaphoreType.DMA,
    ],
)


def _tc_sum_kernel(x_ref, out_ref, acc_ref):
    i = pl.program_id(0)
    n = pl.num_programs(0)

    @pl.when(i == 0)
    def _init():
        acc_ref[...] = jnp.zeros_like(acc_ref)

    acc_ref[...] += jnp.sum(x_ref[...], axis=1)

    @pl.when(i == n - 1)
    def _fin():
        out_ref[...] = acc_ref[...]


def _combine_kernel(t_ref, p_ref, w_ref, b_ref, out_ref):
    s = t_ref[...] + jnp.sum(p_ref[...], axis=1)
    logits = jax.lax.dot_general(
        s, w_ref[...],
        dimension_numbers=(((1,), (1,)), ((), ())),
        preferred_element_type=jnp.float32,
    ) + b_ref[...]
    out_ref[...] = jnp.argmax(logits, axis=1).astype(jnp.int32)[None, :]


def kernel(x, W, b):
    sc_partials = _sc_sum_call(x.reshape(B * S, D))
    tc_partial = pl.pallas_call(
        _tc_sum_kernel,
        grid=(S_TC // CHUNK,),
        in_specs=[pl.BlockSpec((B, CHUNK, D), lambda i: (0, i, 0))],
        out_specs=pl.BlockSpec((B, D), lambda i: (0, 0)),
        out_shape=jax.ShapeDtypeStruct((B, D), jnp.float32),
        scratch_shapes=[pltpu.VMEM((B, D), jnp.float32)],
    )(x)
    out = pl.pallas_call(
        _combine_kernel,
        in_specs=[
            pl.BlockSpec((B, D), lambda: (0, 0)),
            pl.BlockSpec((B, WPB, D), lambda: (0, 0, 0)),
            pl.BlockSpec((E, D), lambda: (0, 0)),
            pl.BlockSpec((1, E), lambda: (0, 0)),
        ],
        out_specs=pl.BlockSpec((1, B), lambda: (0, 0)),
        out_shape=jax.ShapeDtypeStruct((1, B), jnp.int32),
    )(tc_partial, sc_partials.reshape(B, WPB, D), W, b.reshape(1, E))
    return out.reshape(B)


# speedup vs baseline: 1.0047x; 1.0047x over previous
import jax
import jax.numpy as jnp
from jax import lax
from jax.experimental import pallas as pl
from jax.experimental.pallas import tpu as pltpu
from jax.experimental.pallas import tpu_sc as plsc

B, S, D, E = 4, 8192, 2048, 64
VEC = 16
NC, NS = 2, 16
NW = NC * NS
WPB = NW // B

S_TC = 5120
CHUNK = 256
S_SC = S - S_TC
RPW = S_SC // WPB
R = 16
NCHUNK = RPW // R
NG = NCHUNK // 2


def _sc_sum(x_hbm, out_hbm, accA, accB, acc, idxA, idxB, semA, semB):
    wid = lax.axis_index("s") * NC + lax.axis_index("c")
    batch = wid // WPB
    slot = wid % WPB
    base = batch * S + S_TC + slot * RPW
    iota = lax.iota(jnp.int32, VEC)

    def set_start(c, acc_ref, idx_ref, sem, add):
        idx_ref[...] = base + c * R + iota
        pltpu.async_copy(x_hbm.at[idx_ref], acc_ref, sem, add=add)

    def wait(acc_ref, sem):
        pltpu.make_async_copy(x_hbm.at[idxA], acc_ref, sem).wait()

    set_start(0, accA, idxA, semA, False)
    set_start(1, accB, idxB, semB, False)

    def body(g, _):
        wait(accA, semA)
        set_start(2 * g, accA, idxA, semA, True)
        wait(accB, semB)
        set_start(2 * g + 1, accB, idxB, semB, True)
        return 0

    lax.fori_loop(1, NG, body, 0)
    wait(accA, semA)
    wait(accB, semB)

    @plsc.parallel_loop(0, D // VEC, unroll=2)
    def _red(j):
        col = j * VEC
        v = accA[0, pl.ds(col, VEC)]
        for r in range(1, R):
            v = v + accA[r, pl.ds(col, VEC)]
        for r in range(R):
            v = v + accB[r, pl.ds(col, VEC)]
        acc[0, pl.ds(col, VEC)] = v

    pltpu.sync_copy(acc, out_hbm.at[pl.ds(wid, 1)])


_sc_sum_call = pl.kernel(
    _sc_sum,
    out_type=jax.ShapeDtypeStruct((NW, D), jnp.float32),
    mesh=plsc.VectorSubcoreMesh(core_axis_name="c", subcore_axis_name="s"),
    scratch_types=[
        pltpu.VMEM((R, D), jnp.float32),
        pltpu.VMEM((R, D), jnp.float32),
        pltpu.VMEM((1, D), jnp.float32),
        pltpu.VMEM((VEC,), jnp.int32),
        pltpu.VMEM((VEC,), jnp.int32),
        pltpu.SemaphoreType.DMA,
        pltpu.SemaphoreType.DMA,
    ],
)


def _tc_sum_kernel(x_ref, out_ref, acc_ref):
    i = pl.program_id(0)
    n = pl.num_programs(0)

    @pl.when(i == 0)
    def _init():
        acc_ref[...] = jnp.zeros_like(acc_ref)

    acc_ref[...] += jnp.sum(x_ref[...], axis=1)

    @pl.when(i == n - 1)
    def _fin():
        out_ref[...] = acc_ref[...]


def _combine_kernel(t_ref, p_ref, w_ref, b_ref, out_ref):
    s = t_ref[...] + jnp.sum(p_ref[...], axis=1)
    logits = jax.lax.dot_general(
        s, w_ref[...],
        dimension_numbers=(((1,), (1,)), ((), ())),
        preferred_element_type=jnp.float32,
    ) + b_ref[...]
    out_ref[...] = jnp.argmax(logits, axis=1).astype(jnp.int32)[None, :]


def kernel(x, W, b):
    sc_partials = _sc_sum_call(x.reshape(B * S, D))
    tc_partial = pl.pallas_call(
        _tc_sum_kernel,
        grid=(S_TC // CHUNK,),
        in_specs=[pl.BlockSpec((B, CHUNK, D), lambda i: (0, i, 0))],
        out_specs=pl.BlockSpec((B, D), lambda i: (0, 0)),
        out_shape=jax.ShapeDtypeStruct((B, D), jnp.float32),
        scratch_shapes=[pltpu.VMEM((B, D), jnp.float32)],
    )(x)
    out = pl.pallas_call(
        _combine_kernel,
        in_specs=[
            pl.BlockSpec((B, D), lambda: (0, 0)),
            pl.BlockSpec((B, WPB, D), lambda: (0, 0, 0)),
            pl.BlockSpec((E, D), lambda: (0, 0)),
            pl.BlockSpec((1, E), lambda: (0, 0)),
        ],
        out_specs=pl.BlockSpec((1, B), lambda: (0, 0)),
        out_shape=jax.ShapeDtypeStruct((1, B), jnp.int32),
    )(tc_partial, sc_partials.reshape(B, WPB, D), W, b.reshape(1, E))
    return out.reshape(B)


# speedup vs baseline: 1.1664x; 1.1610x over previous
import jax
import jax.numpy as jnp
from jax.experimental import pallas as pl
from jax.experimental.pallas import tpu as pltpu

B, S, D, E = 4, 8192, 2048, 64
CHUNK = 128
NCHKS = S // CHUNK


def _tc_sum_kernel(x_ref, out_ref):
    out_ref[...] = jnp.sum(x_ref[...], axis=1)[None]


def _combine_kernel(p_ref, w_ref, b_ref, out_ref):
    s = jnp.sum(p_ref[...], axis=0)
    logits = jax.lax.dot_general(
        s, w_ref[...],
        dimension_numbers=(((1,), (1,)), ((), ())),
        preferred_element_type=jnp.float32,
    ) + b_ref[...]
    out_ref[...] = jnp.argmax(logits, axis=1).astype(jnp.int32)[None, :]


def kernel(x, W, b):
    partials = pl.pallas_call(
        _tc_sum_kernel,
        grid=(NCHKS,),
        in_specs=[pl.BlockSpec((B, CHUNK, D), lambda i: (0, i, 0))],
        out_specs=pl.BlockSpec((1, B, D), lambda i: (i, 0, 0)),
        out_shape=jax.ShapeDtypeStruct((NCHKS, B, D), jnp.float32),
    )(x)
    out = pl.pallas_call(
        _combine_kernel,
        in_specs=[
            pl.BlockSpec((NCHKS, B, D), lambda: (0, 0, 0)),
            pl.BlockSpec((E, D), lambda: (0, 0)),
            pl.BlockSpec((1, E), lambda: (0, 0)),
        ],
        out_specs=pl.BlockSpec((1, B), lambda: (0, 0)),
        out_shape=jax.ShapeDtypeStruct((1, B), jnp.int32),
    )(partials, W, b.reshape(1, E))
    return out.reshape(B)
